# baseline (speedup 1.0000x reference)
import jax
import jax.numpy as jnp
from jax import lax
from jax.experimental import pallas as pl
from jax.experimental.pallas import tpu as pltpu
from jax.experimental.pallas import tpu_sc as plsc

N = 10000
E = 320000
D = 128
D_OUT = 8
K_LAYERS = 10
ALPHA = 0.1

NTILES = 16
RPT = 640
NPAD = NTILES * RPT
EPT = E // NTILES

CH = 48
ECH = 432
IB = 16
NBLK = ECH // IB
UCH = 32
RCHU = RPT // UCH


def _rsqrt16(x):
    xi = lax.bitcast_convert_type(x, jnp.int32)
    yi = jnp.int32(0x5F3759DF) - (xi >> 1)
    y = lax.bitcast_convert_type(yi, jnp.float32)
    for _ in range(3):
        y = y * (1.5 - 0.5 * x * y * y)
    return y


def _sc_propagate_body(h, srcidx, dstidx, z_out, hn, hn0, nrm, amid, blast,
                       agg, sidxb, didxb, buf0, buf1, buf2, buf3, aux16,
                       bux16, gs0, gs1, gs2, gs3, ss0, ss1, ss2, ss3):
    c = lax.axis_index("c")
    s = lax.axis_index("s")
    wid = c * NTILES + s
    srcv = srcidx.at[wid]
    dstv = dstidx.at[wid]

    def _fill(ref, rows, val):
        def _f(i, _):
            for g in range(8):
                ref[i, pl.ds(g * 16, 16)] = jnp.full((16,), val, jnp.float32)
            return 0
        lax.fori_loop(0, rows, _f, 0)

    _fill(buf2, CH, 0.0)
    _fill(buf1, CH, 1.0)

    for rc in range(RCHU):
        pltpu.sync_copy(buf2.at[pl.ds(0, UCH)],
                        agg.at[pl.ds(s * RPT + rc * UCH, UCH)])
    plsc.subcore_barrier()

    def _degblk(blk, _):
        pltpu.sync_copy(dstv.at[pl.ds(blk * IB, IB)], didxb)
        descs = []
        for j in range(IB):
            descs.append(pltpu.async_copy(buf1, agg.at[didxb.at[j]],
                                          ss0, add=True))
            if j >= 4:
                descs[j - 4].wait()
        for d in descs[IB - 4:]:
            d.wait()
        return 0
    lax.fori_loop(0, NBLK, _degblk, 0)
    plsc.subcore_barrier()

    for rc in range(RCHU):
        r0 = s * RPT + rc * UCH
        flat = c * NPAD + r0
        pltpu.sync_copy(agg.at[pl.ds(r0, UCH)], buf0.at[pl.ds(0, UCH)])

        def _coef(i, _):
            d = jnp.maximum(buf0[i, pl.ds(0, 16)], 1.0)
            n = _rsqrt16(d)
            aux16[i, :] = n
            bux16[i, :] = (1.0 - ALPHA) * n * n
            return 0
        lax.fori_loop(0, UCH, _coef, 0)
        pltpu.sync_copy(aux16, nrm.at[pl.ds(flat, UCH)])
        pltpu.sync_copy(bux16, amid.at[pl.ds(flat, UCH)])

        def _coefc(i, _):
            d = jnp.maximum(buf0[i, pl.ds(0, 16)], 1.0)
            bux16[i, :] = ALPHA * d * aux16[i, :]
            return 0
        lax.fori_loop(0, UCH, _coefc, 0)
        pltpu.sync_copy(bux16, blast.at[pl.ds(flat, UCH)])
        pltpu.sync_copy(buf2.at[pl.ds(0, UCH)], agg.at[pl.ds(r0, UCH)])
        pltpu.sync_copy(h.at[pl.ds(r0, UCH)], buf3.at[pl.ds(0, UCH)])

        def _scale(i, _):
            nv = aux16[i, :]
            for g in range(8):
                sl = pl.ds(g * 16, 16)
                buf3[i, sl] = buf3[i, sl] * nv
            return 0
        lax.fori_loop(0, UCH, _scale, 0)
        pltpu.sync_copy(buf3.at[pl.ds(0, UCH)], hn.at[pl.ds(flat, UCH)])
        pltpu.sync_copy(buf3.at[pl.ds(0, UCH)], hn0.at[pl.ds(flat, UCH)])
    plsc.subcore_barrier()

    bufs = (buf0, buf1, buf2, buf3)
    gsems = (gs0, gs1, gs2, gs3)
    ssems = (ss0, ss1, ss2, ss3)
    NB = 4

    def _gather_scatter():
        def _blk(blk, _):
            pltpu.sync_copy(srcv.at[pl.ds(blk * IB, IB)], sidxb)
            pltpu.sync_copy(dstv.at[pl.ds(blk * IB, IB)], didxb)
            gd = {}
            sd = {}
            gd[0] = pltpu.async_copy(hn.at[sidxb.at[0]], bufs[0], gsems[0])
            gd[1] = pltpu.async_copy(hn.at[sidxb.at[1]], bufs[1], gsems[1])
            for j in range(IB):
                b = j % NB
                gd[j].wait()
                sd[j] = pltpu.async_copy(bufs[b], agg.at[didxb.at[j]],
                                         ssems[b], add=True)
                if j + 2 < IB:
                    if j >= 2:
                        sd[j - 2].wait()
                    nb = (j + 2) % NB
                    gd[j + 2] = pltpu.async_copy(hn.at[sidxb.at[j + 2]],
                                                 bufs[nb], gsems[nb])
            for j in range(IB - 4, IB):
                sd[j].wait()
            return 0
        lax.fori_loop(0, NBLK, _blk, 0)

    def _update(is_last):
        if not is_last:
            _fill(buf2, UCH, 0.0)
        wd = None
        zds = []
        for rc in range(RCHU):
            r0 = s * RPT + rc * UCH
            flat = c * NPAD + r0
            if wd is not None:
                wd.wait()
            d0 = pltpu.async_copy(agg.at[pl.ds(r0, UCH)],
                                  buf0.at[pl.ds(0, UCH)], gs0)
            d1 = pltpu.async_copy(hn0.at[pl.ds(flat, UCH)],
                                  buf1.at[pl.ds(0, UCH)], gs1)
            if is_last:
                d2 = pltpu.async_copy(nrm.at[pl.ds(flat, UCH)], aux16, gs2)
                d3 = pltpu.async_copy(blast.at[pl.ds(flat, UCH)], bux16, gs3)
                d3.wait()
            else:
                d2 = pltpu.async_copy(amid.at[pl.ds(flat, UCH)], aux16, gs2)
            d0.wait()
            d1.wait()
            d2.wait()

            def _ubody(i, _):
                if is_last:
                    a = (1.0 - ALPHA) * aux16[i, :]
                    b = bux16[i, :]
                else:
                    a = aux16[i, :]
                for g in range(8):
                    sl = pl.ds(g * 16, 16)
                    acc = a * buf0[i, sl]
                    if is_last:
                        acc = acc + b * buf1[i, sl]
                    else:
                        acc = acc + ALPHA * buf1[i, sl]
                    buf0[i, sl] = acc
                return 0
            lax.fori_loop(0, UCH, _ubody, 0)
            if is_last:
                wd = pltpu.async_copy(buf0.at[pl.ds(0, UCH)],
                                      z_out.at[pl.ds(flat, UCH)], ss0)
            else:
                wd = pltpu.async_copy(buf0.at[pl.ds(0, UCH)],
                                      hn.at[pl.ds(flat, UCH)], ss0)
                zds.append(pltpu.async_copy(
                    buf2.at[pl.ds(0, UCH)], agg.at[pl.ds(r0, UCH)], ss2))
        wd.wait()
        for zd in zds:
            zd.wait()

    def _layer(k, _):
        _gather_scatter()
        plsc.subcore_barrier()
        _update(False)
        plsc.subcore_barrier()
        return 0
    lax.fori_loop(0, K_LAYERS - 1, _layer, 0)
    _gather_scatter()
    plsc.subcore_barrier()
    _update(True)


def _sc_propagate(h_pad, srcidx, dstidx):
    f32 = jnp.float32
    mesh = plsc.VectorSubcoreMesh(core_axis_name="c", subcore_axis_name="s")
    kfn = pl.kernel(
        _sc_propagate_body,
        out_type=[
            jax.ShapeDtypeStruct((2 * NPAD, D), f32),
            jax.ShapeDtypeStruct((2 * NPAD, D), f32),
            jax.ShapeDtypeStruct((2 * NPAD, D), f32),
            jax.ShapeDtypeStruct((2 * NPAD, 16), f32),
            jax.ShapeDtypeStruct((2 * NPAD, 16), f32),
            jax.ShapeDtypeStruct((2 * NPAD, 16), f32),
        ],
        mesh=mesh,
        scratch_types=[
            pltpu.VMEM_SHARED((NPAD, D), f32),
            pltpu.VMEM((IB, CH), jnp.int32),
            pltpu.VMEM((IB, CH), jnp.int32),
            pltpu.VMEM((CH, D), f32),
            pltpu.VMEM((CH, D), f32),
            pltpu.VMEM((CH, D), f32),
            pltpu.VMEM((CH, D), f32),
            pltpu.VMEM((UCH, 16), f32),
            pltpu.VMEM((UCH, 16), f32),
            pltpu.SemaphoreType.DMA,
            pltpu.SemaphoreType.DMA,
            pltpu.SemaphoreType.DMA,
            pltpu.SemaphoreType.DMA,
            pltpu.SemaphoreType.DMA,
            pltpu.SemaphoreType.DMA,
            pltpu.SemaphoreType.DMA,
            pltpu.SemaphoreType.DMA,
        ],
    )
    z, _, _, _, _, _ = kfn(h_pad, srcidx, dstidx)
    return z


def _prep_edges(edge_index, graph_id):
    src = edge_index[0].astype(jnp.int32) + graph_id * NPAD
    dst = edge_index[1].astype(jnp.int32)
    pad = ECH * CH - EPT
    src = jnp.pad(src.reshape(NTILES, EPT), ((0, 0), (0, pad)),
                  constant_values=graph_id * NPAD + N)
    dst = jnp.pad(dst.reshape(NTILES, EPT), ((0, 0), (0, pad)),
                  constant_values=N)
    return src.reshape(NTILES, ECH, CH), dst.reshape(NTILES, ECH, CH)


def _tc_matmul(x_pad, Wt):
    def body(x_ref, w_ref, o_ref):
        o_ref[...] = jnp.dot(x_ref[...], w_ref[...],
                             preferred_element_type=jnp.float32)
    return pl.pallas_call(
        body, out_shape=jax.ShapeDtypeStruct((NPAD, D), jnp.float32),
    )(x_pad, Wt)


def _tc_attention_beta(z_flat, Wa1, ba1_2d, Wa2):
    def body(z_ref, wa1_ref, ba1_ref, wa2_ref, beta_ref):
        wa1 = wa1_ref[...]
        ba1 = ba1_ref[...]
        wa2 = wa2_ref[...]
        mask = lax.broadcasted_iota(jnp.int32, (NPAD, 1), 0) < N
        s = []
        for m in range(2):
            zm = z_ref[m * NPAD:(m + 1) * NPAD, :]
            t = jnp.tanh(jnp.dot(zm, wa1, preferred_element_type=jnp.float32)
                         + ba1)
            t = jnp.dot(t, wa2, preferred_element_type=jnp.float32)
            s.append(jnp.sum(jnp.where(mask, t, 0.0)) / N)
        mx = jnp.maximum(s[0], s[1])
        e0 = jnp.exp(s[0] - mx)
        e1 = jnp.exp(s[1] - mx)
        den = e0 + e1
        ones = jnp.ones((1, D), jnp.float32)
        beta_ref[0:1, :] = (e0 / den) * ones
        beta_ref[1:2, :] = (e1 / den) * ones
    return pl.pallas_call(
        body, out_shape=jax.ShapeDtypeStruct((2, D), jnp.float32),
    )(z_flat, Wa1, ba1_2d, Wa2)


def _tc_combine(z_flat, beta, Wp, bp_2d):
    def body(z_ref, beta_ref, wp_ref, bp_ref, h_ref, lg_ref):
        h = (z_ref[0:NPAD, :] * beta_ref[0:1, :]
             + z_ref[NPAD:2 * NPAD, :] * beta_ref[1:2, :])
        h_ref[...] = h
        lg_ref[...] = jnp.dot(h, wp_ref[...],
                              preferred_element_type=jnp.float32) + bp_ref[...]
    return pl.pallas_call(
        body,
        out_shape=[
            jax.ShapeDtypeStruct((NPAD, D), jnp.float32),
            jax.ShapeDtypeStruct((NPAD, D_OUT), jnp.float32),
        ],
    )(z_flat, beta, Wp, bp_2d)


def kernel(x, edge_index0, edge_index1, Wt, Wa1, ba1, Wa2, Wp, bp):
    x_pad = jnp.pad(x, ((0, NPAD - N), (0, 0)))
    h_pad = _tc_matmul(x_pad, Wt)

    s0, d0 = _prep_edges(edge_index0, 0)
    s1, d1 = _prep_edges(edge_index1, 1)
    srcidx = jnp.concatenate([s0, s1], axis=0)
    dstidx = jnp.concatenate([d0, d1], axis=0)

    z_flat = _sc_propagate(h_pad, srcidx, dstidx)

    beta = _tc_attention_beta(z_flat, Wa1, ba1.reshape(1, D), Wa2)
    h_out, logits = _tc_combine(z_flat, beta, Wp, bp.reshape(1, D_OUT))
    return (logits[:N], h_out[:N])


# speedup vs baseline: 1.4531x; 1.4531x over previous
import jax
import jax.numpy as jnp
from jax import lax
from jax.experimental import pallas as pl
from jax.experimental.pallas import tpu as pltpu
from jax.experimental.pallas import tpu_sc as plsc

N = 10000
E = 320000
D = 128
D_OUT = 8
K_LAYERS = 10
ALPHA = 0.1

NTILES = 16
RPT = 640
NPAD = NTILES * RPT
EPT = E // NTILES

CH = 64
ECH = 320
IB = 32
NBLK = ECH // IB
ECH_PF = ECH
UCH = 64
RCHU = RPT // UCH


def _rsqrt16(x):
    xi = lax.bitcast_convert_type(x, jnp.int32)
    yi = jnp.int32(0x5F3759DF) - (xi >> 1)
    y = lax.bitcast_convert_type(yi, jnp.float32)
    for _ in range(3):
        y = y * (1.5 - 0.5 * x * y * y)
    return y


def _sc_propagate_body(h, srcidx, dstidx, z_out, hn, hn0, nrm, amid, blast,
                       agg, sidx0, didx0, buf0, buf1, buf2,
                       aux16, bux16, gs0, gs1, gs2, ss0, ss1, ss2, isrc,
                       idst):
    c = lax.axis_index("c")
    s = lax.axis_index("s")
    wid = c * NTILES + s
    srcv = srcidx.at[wid]
    dstv = dstidx.at[wid]
    bufs = (buf0, buf1, buf2)
    gsems = (gs0, gs1, gs2)
    ssems = (ss0, ss1, ss2)

    def _fill(ref, rows, val):
        def _f(i, _):
            for g in range(8):
                ref[i, pl.ds(g * 16, 16)] = jnp.full((16,), val, jnp.float32)
            return 0
        lax.fori_loop(0, rows, _f, 0)

    def _zero_agg_chunk(r0, sync=True, sem=None, out=None):
        if sync:
            pltpu.sync_copy(buf2, agg.at[pl.ds(r0, UCH)])
        else:
            out.append(pltpu.async_copy(buf2, agg.at[pl.ds(r0, UCH)], sem))

    _fill(buf2, CH, 0.0)
    _fill(buf1, CH, 1.0)

    for rc in range(RCHU):
        _zero_agg_chunk(s * RPT + rc * UCH)
    plsc.subcore_barrier()

    def _degblk(blk, _):
        pltpu.sync_copy(dstv.at[pl.ds(blk * IB, IB)], didx0)
        descs = []
        for j in range(IB):
            descs.append(pltpu.async_copy(buf1, agg.at[didx0.at[j]],
                                          ss0, add=True))
            if j >= 4:
                descs[j - 4].wait()
        for d in descs[IB - 4:]:
            d.wait()
        return 0
    lax.fori_loop(0, NBLK, _degblk, 0)
    plsc.subcore_barrier()

    for rc in range(RCHU):
        r0 = s * RPT + rc * UCH
        flat = c * NPAD + r0
        pltpu.sync_copy(agg.at[pl.ds(r0, UCH)], buf0.at[pl.ds(0, UCH)])

        def _coef(i, _):
            d = jnp.maximum(buf0[i, pl.ds(0, 16)], 1.0)
            n = _rsqrt16(d)
            aux16[i, :] = n
            bux16[i, :] = (1.0 - ALPHA) * n * n
            return 0
        lax.fori_loop(0, UCH, _coef, 0)
        pltpu.sync_copy(aux16, nrm.at[pl.ds(flat, UCH)])
        pltpu.sync_copy(bux16, amid.at[pl.ds(flat, UCH)])

        def _coefc(i, _):
            d = jnp.maximum(buf0[i, pl.ds(0, 16)], 1.0)
            bux16[i, :] = ALPHA * d * aux16[i, :]
            return 0
        lax.fori_loop(0, UCH, _coefc, 0)
        pltpu.sync_copy(bux16, blast.at[pl.ds(flat, UCH)])
        _zero_agg_chunk(r0)
        pltpu.sync_copy(h.at[pl.ds(r0, UCH)], buf0.at[pl.ds(0, UCH)])

        def _scale(i, _):
            nv = aux16[i, :]
            for g in range(8):
                sl = pl.ds(g * 16, 16)
                buf0[i, sl] = buf0[i, sl] * nv
            return 0
        lax.fori_loop(0, UCH, _scale, 0)
        pltpu.sync_copy(buf0.at[pl.ds(0, UCH)], hn.at[pl.ds(flat, UCH)])
        pltpu.sync_copy(buf0.at[pl.ds(0, UCH)], hn0.at[pl.ds(flat, UCH)])
    plsc.subcore_barrier()

    def _gather_scatter():
        def _blk(blk, _):
            pltpu.sync_copy(srcv.at[pl.ds(blk * IB, IB)], sidx0)
            pltpu.sync_copy(dstv.at[pl.ds(blk * IB, IB)], didx0)
            gd = {}
            sd = {}
            gd[0] = pltpu.async_copy(hn.at[sidx0.at[0]], bufs[0], gsems[0])
            gd[1] = pltpu.async_copy(hn.at[sidx0.at[1]], bufs[1], gsems[1])
            for j in range(IB):
                b = j % 3
                gd[j].wait()
                sd[j] = pltpu.async_copy(bufs[b], agg.at[didx0.at[j]],
                                         ssems[b], add=True)
                if j + 2 < IB:
                    if j >= 1:
                        sd[j - 1].wait()
                    nb = (j + 2) % 3
                    gd[j + 2] = pltpu.async_copy(hn.at[sidx0.at[j + 2]],
                                                 bufs[nb], gsems[nb])
            for j in range(IB - 3, IB):
                sd[j].wait()
            return 0
        lax.fori_loop(0, NBLK, _blk, 0)

    def _update(is_last):
        if not is_last:
            _fill(buf2, CH, 0.0)
        wd = None
        zds = []
        for rc in range(RCHU):
            r0 = s * RPT + rc * UCH
            flat = c * NPAD + r0
            if wd is not None:
                wd.wait()
            d0 = pltpu.async_copy(agg.at[pl.ds(r0, UCH)],
                                  buf0.at[pl.ds(0, UCH)], gs0)
            d1 = pltpu.async_copy(hn0.at[pl.ds(flat, UCH)],
                                  buf1.at[pl.ds(0, UCH)], gs1)
            if is_last:
                d2 = pltpu.async_copy(nrm.at[pl.ds(flat, UCH)], aux16, isrc)
                d3 = pltpu.async_copy(blast.at[pl.ds(flat, UCH)], bux16, idst)
                d3.wait()
            else:
                d2 = pltpu.async_copy(amid.at[pl.ds(flat, UCH)], aux16, isrc)
            d0.wait()
            d1.wait()
            d2.wait()

            def _ubody(i, _):
                if is_last:
                    a = (1.0 - ALPHA) * aux16[i, :]
                    b = bux16[i, :]
                else:
                    a = aux16[i, :]
                for g in range(8):
                    sl = pl.ds(g * 16, 16)
                    acc = a * buf0[i, sl]
                    if is_last:
                        acc = acc + b * buf1[i, sl]
                    else:
                        acc = acc + ALPHA * buf1[i, sl]
                    buf1[i, sl] = acc
                return 0
            lax.fori_loop(0, UCH, _ubody, 0)
            if is_last:
                wd = pltpu.async_copy(buf1.at[pl.ds(0, UCH)],
                                      z_out.at[pl.ds(flat, UCH)], ss0)
            else:
                wd = pltpu.async_copy(buf1.at[pl.ds(0, UCH)],
                                      hn.at[pl.ds(flat, UCH)], ss0)
                _zero_agg_chunk(r0, sync=False, sem=ss1, out=zds)
        wd.wait()
        for zd in zds:
            zd.wait()

    def _layer(k, _):
        _gather_scatter()
        plsc.subcore_barrier()

        @pl.when(k < K_LAYERS - 1)
        def _mid():
            _update(False)

        @pl.when(k == K_LAYERS - 1)
        def _last():
            _update(True)
        plsc.subcore_barrier()
        return 0
    lax.fori_loop(0, K_LAYERS, _layer, 0)


def _sc_propagate(h_pad, srcidx, dstidx):
    f32 = jnp.float32
    mesh = plsc.VectorSubcoreMesh(core_axis_name="c", subcore_axis_name="s")
    kfn = pl.kernel(
        _sc_propagate_body,
        out_type=[
            jax.ShapeDtypeStruct((2 * NPAD, D), f32),
            jax.ShapeDtypeStruct((2 * NPAD, D), f32),
            jax.ShapeDtypeStruct((2 * NPAD, D), f32),
            jax.ShapeDtypeStruct((2 * NPAD, 16), f32),
            jax.ShapeDtypeStruct((2 * NPAD, 16), f32),
            jax.ShapeDtypeStruct((2 * NPAD, 16), f32),
        ],
        mesh=mesh,
        scratch_types=[
            pltpu.VMEM_SHARED((NPAD, D), f32),
            pltpu.VMEM((IB, CH), jnp.int32),
            pltpu.VMEM((IB, CH), jnp.int32),
            pltpu.VMEM((CH, D), f32),
            pltpu.VMEM((CH, D), f32),
            pltpu.VMEM((CH, D), f32),
            pltpu.VMEM((UCH, 16), f32),
            pltpu.VMEM((UCH, 16), f32),
            pltpu.SemaphoreType.DMA,
            pltpu.SemaphoreType.DMA,
            pltpu.SemaphoreType.DMA,
            pltpu.SemaphoreType.DMA,
            pltpu.SemaphoreType.DMA,
            pltpu.SemaphoreType.DMA,
            pltpu.SemaphoreType.DMA,
            pltpu.SemaphoreType.DMA,
        ],
    )
    z, _, _, _, _, _ = kfn(h_pad, srcidx, dstidx)
    return z


def _prep_edges(edge_index, graph_id):
    src = edge_index[0].astype(jnp.int32) + graph_id * NPAD
    dst = edge_index[1].astype(jnp.int32)
    pad = ECH_PF * CH - EPT
    src = jnp.pad(src.reshape(NTILES, EPT), ((0, 0), (0, pad)),
                  constant_values=graph_id * NPAD + N)
    dst = jnp.pad(dst.reshape(NTILES, EPT), ((0, 0), (0, pad)),
                  constant_values=N)
    return src.reshape(NTILES, ECH_PF, CH), dst.reshape(NTILES, ECH_PF, CH)


def _tc_matmul(x_pad, Wt):
    def body(x_ref, w_ref, o_ref):
        o_ref[...] = jnp.dot(x_ref[...], w_ref[...],
                             preferred_element_type=jnp.float32)
    return pl.pallas_call(
        body, out_shape=jax.ShapeDtypeStruct((NPAD, D), jnp.float32),
    )(x_pad, Wt)


def _tc_attention_beta(z_flat, Wa1, ba1_2d, Wa2):
    def body(z_ref, wa1_ref, ba1_ref, wa2_ref, beta_ref):
        wa1 = wa1_ref[...]
        ba1 = ba1_ref[...]
        wa2 = wa2_ref[...]
        mask = lax.broadcasted_iota(jnp.int32, (NPAD, 1), 0) < N
        s = []
        for m in range(2):
            zm = z_ref[m * NPAD:(m + 1) * NPAD, :]
            t = jnp.tanh(jnp.dot(zm, wa1, preferred_element_type=jnp.float32)
                         + ba1)
            t = jnp.dot(t, wa2, preferred_element_type=jnp.float32)
            s.append(jnp.sum(jnp.where(mask, t, 0.0)) / N)
        mx = jnp.maximum(s[0], s[1])
        e0 = jnp.exp(s[0] - mx)
        e1 = jnp.exp(s[1] - mx)
        den = e0 + e1
        ones = jnp.ones((1, D), jnp.float32)
        beta_ref[0:1, :] = (e0 / den) * ones
        beta_ref[1:2, :] = (e1 / den) * ones
    return pl.pallas_call(
        body, out_shape=jax.ShapeDtypeStruct((2, D), jnp.float32),
    )(z_flat, Wa1, ba1_2d, Wa2)


def _tc_combine(z_flat, beta, Wp, bp_2d):
    def body(z_ref, beta_ref, wp_ref, bp_ref, h_ref, lg_ref):
        h = (z_ref[0:NPAD, :] * beta_ref[0:1, :]
             + z_ref[NPAD:2 * NPAD, :] * beta_ref[1:2, :])
        h_ref[...] = h
        lg_ref[...] = jnp.dot(h, wp_ref[...],
                              preferred_element_type=jnp.float32) + bp_ref[...]
    return pl.pallas_call(
        body,
        out_shape=[
            jax.ShapeDtypeStruct((NPAD, D), jnp.float32),
            jax.ShapeDtypeStruct((NPAD, D_OUT), jnp.float32),
        ],
    )(z_flat, beta, Wp, bp_2d)


def kernel(x, edge_index0, edge_index1, Wt, Wa1, ba1, Wa2, Wp, bp):
    x_pad = jnp.pad(x, ((0, NPAD - N), (0, 0)))
    h_pad = _tc_matmul(x_pad, Wt)

    s0, d0 = _prep_edges(edge_index0, 0)
    s1, d1 = _prep_edges(edge_index1, 1)
    srcidx = jnp.concatenate([s0, s1], axis=0)
    dstidx = jnp.concatenate([d0, d1], axis=0)

    z_flat = _sc_propagate(h_pad, srcidx, dstidx)

    beta = _tc_attention_beta(z_flat, Wa1, ba1.reshape(1, D), Wa2)
    h_out, logits = _tc_combine(z_flat, beta, Wp, bp.reshape(1, D_OUT))
    return (logits[:N], h_out[:N])


# speedup vs baseline: 1.4782x; 1.0172x over previous
import jax
import jax.numpy as jnp
from jax import lax
from jax.experimental import pallas as pl
from jax.experimental.pallas import tpu as pltpu
from jax.experimental.pallas import tpu_sc as plsc

N = 10000
E = 320000
D = 128
D_OUT = 8
K_LAYERS = 10
ALPHA = 0.1

NTILES = 16
RPT = 640
NPAD = NTILES * RPT
EPT = E // NTILES

CH = 64
ECH = 320
IB = 16
NBLK = ECH // (2 * IB)
ECH_PF = ECH + IB
UCH = 64
RCHU = RPT // UCH


def _rsqrt16(x):
    xi = lax.bitcast_convert_type(x, jnp.int32)
    yi = jnp.int32(0x5F3759DF) - (xi >> 1)
    y = lax.bitcast_convert_type(yi, jnp.float32)
    for _ in range(3):
        y = y * (1.5 - 0.5 * x * y * y)
    return y


def _sc_propagate_body(h, srcidx, dstidx, z_out, hn, hn0, nrm, amid, blast,
                       agg, sidx0, sidx1, didx0, didx1, buf0, buf1, buf2,
                       aux16, bux16, gs0, gs1, gs2, ss0, ss1, ss2, isrc,
                       idst):
    c = lax.axis_index("c")
    s = lax.axis_index("s")
    wid = c * NTILES + s
    srcv = srcidx.at[wid]
    dstv = dstidx.at[wid]
    bufs = (buf0, buf1, buf2)
    gsems = (gs0, gs1, gs2)
    ssems = (ss0, ss1, ss2)

    def _fill(ref, rows, val):
        def _f(i, _):
            for g in range(8):
                ref[i, pl.ds(g * 16, 16)] = jnp.full((16,), val, jnp.float32)
            return 0
        lax.fori_loop(0, rows, _f, 0)

    def _zero_agg_chunk(r0, sync=True, sem=None, out=None):
        if sync:
            pltpu.sync_copy(buf2, agg.at[pl.ds(r0, UCH)])
        else:
            out.append(pltpu.async_copy(buf2, agg.at[pl.ds(r0, UCH)], sem))

    _fill(buf2, CH, 0.0)
    _fill(buf1, CH, 1.0)

    for rc in range(RCHU):
        _zero_agg_chunk(s * RPT + rc * UCH)
    plsc.subcore_barrier()

    def _degblk(blk, _):
        pltpu.sync_copy(dstv.at[pl.ds(blk * IB, IB)], didx0)
        descs = []
        for j in range(IB):
            descs.append(pltpu.async_copy(buf1, agg.at[didx0.at[j]],
                                          ss0, add=True))
            if j >= 4:
                descs[j - 4].wait()
        for d in descs[IB - 4:]:
            d.wait()
        return 0
    lax.fori_loop(0, 2 * NBLK, _degblk, 0)
    plsc.subcore_barrier()

    for rc in range(RCHU):
        r0 = s * RPT + rc * UCH
        flat = c * NPAD + r0
        pltpu.sync_copy(agg.at[pl.ds(r0, UCH)], buf0.at[pl.ds(0, UCH)])

        def _coef(i, _):
            d = jnp.maximum(buf0[i, pl.ds(0, 16)], 1.0)
            n = _rsqrt16(d)
            aux16[i, :] = n
            bux16[i, :] = (1.0 - ALPHA) * n * n
            return 0
        lax.fori_loop(0, UCH, _coef, 0)
        pltpu.sync_copy(aux16, nrm.at[pl.ds(flat, UCH)])
        pltpu.sync_copy(bux16, amid.at[pl.ds(flat, UCH)])

        def _coefc(i, _):
            d = jnp.maximum(buf0[i, pl.ds(0, 16)], 1.0)
            bux16[i, :] = ALPHA * d * aux16[i, :]
            return 0
        lax.fori_loop(0, UCH, _coefc, 0)
        pltpu.sync_copy(bux16, blast.at[pl.ds(flat, UCH)])
        _zero_agg_chunk(r0)
        pltpu.sync_copy(h.at[pl.ds(r0, UCH)], buf0.at[pl.ds(0, UCH)])

        def _scale(i, _):
            nv = aux16[i, :]
            for g in range(8):
                sl = pl.ds(g * 16, 16)
                buf0[i, sl] = buf0[i, sl] * nv
            return 0
        lax.fori_loop(0, UCH, _scale, 0)
        pltpu.sync_copy(buf0.at[pl.ds(0, UCH)], hn.at[pl.ds(flat, UCH)])
        pltpu.sync_copy(buf0.at[pl.ds(0, UCH)], hn0.at[pl.ds(flat, UCH)])
    plsc.subcore_barrier()

    def _gather_scatter():
        pltpu.async_copy(srcv.at[pl.ds(0, IB)], sidx0, isrc)
        pltpu.async_copy(dstv.at[pl.ds(0, IB)], didx0, idst)

        def _body(p, _):
            pltpu.make_async_copy(srcv.at[pl.ds(0, IB)], sidx0, isrc).wait()
            pltpu.make_async_copy(dstv.at[pl.ds(0, IB)], didx0, idst).wait()
            sidxj = lambda j: (sidx0 if j < IB else sidx1).at[j % IB]
            didxj = lambda j: (didx0 if j < IB else didx1).at[j % IB]
            gd = {}
            sd = {}
            rs1 = rd1 = None
            gd[0] = pltpu.async_copy(hn.at[sidxj(0)], bufs[0], gsems[0])
            gd[1] = pltpu.async_copy(hn.at[sidxj(1)], bufs[1], gsems[1])
            for j in range(2 * IB):
                b = j % 3
                gd[j].wait()
                sd[j] = pltpu.async_copy(bufs[b], agg.at[didxj(j)],
                                         ssems[b], add=True)
                if j == 2:
                    rs1 = pltpu.async_copy(
                        srcv.at[pl.ds((2 * p + 1) * IB, IB)], sidx1, isrc)
                    rd1 = pltpu.async_copy(
                        dstv.at[pl.ds((2 * p + 1) * IB, IB)], didx1, idst)
                if j == 13:
                    rs1.wait()
                    rd1.wait()
                if j == 17:
                    pltpu.async_copy(
                        srcv.at[pl.ds((2 * p + 2) * IB, IB)], sidx0, isrc)
                    pltpu.async_copy(
                        dstv.at[pl.ds((2 * p + 2) * IB, IB)], didx0, idst)
                if j + 2 < 2 * IB:
                    if j >= 1:
                        sd[j - 1].wait()
                    nb = (j + 2) % 3
                    gd[j + 2] = pltpu.async_copy(hn.at[sidxj(j + 2)],
                                                 bufs[nb], gsems[nb])
            for j in range(2 * IB - 3, 2 * IB):
                sd[j].wait()
            return 0
        lax.fori_loop(0, NBLK, _body, 0)
        pltpu.make_async_copy(srcv.at[pl.ds(0, IB)], sidx0, isrc).wait()
        pltpu.make_async_copy(dstv.at[pl.ds(0, IB)], didx0, idst).wait()

    def _update(is_last):
        if not is_last:
            _fill(buf2, CH, 0.0)
        wd = None
        zds = []
        for rc in range(RCHU):
            r0 = s * RPT + rc * UCH
            flat = c * NPAD + r0
            if wd is not None:
                wd.wait()
            d0 = pltpu.async_copy(agg.at[pl.ds(r0, UCH)],
                                  buf0.at[pl.ds(0, UCH)], gs0)
            d1 = pltpu.async_copy(hn0.at[pl.ds(flat, UCH)],
                                  buf1.at[pl.ds(0, UCH)], gs1)
            if is_last:
                d2 = pltpu.async_copy(nrm.at[pl.ds(flat, UCH)], aux16, isrc)
                d3 = pltpu.async_copy(blast.at[pl.ds(flat, UCH)], bux16, idst)
                d3.wait()
            else:
                d2 = pltpu.async_copy(amid.at[pl.ds(flat, UCH)], aux16, isrc)
            d0.wait()
            d1.wait()
            d2.wait()

            def _ubody(i, _):
                if is_last:
                    a = (1.0 - ALPHA) * aux16[i, :]
                    b = bux16[i, :]
                else:
                    a = aux16[i, :]
                for g in range(8):
                    sl = pl.ds(g * 16, 16)
                    acc = a * buf0[i, sl]
                    if is_last:
                        acc = acc + b * buf1[i, sl]
                    else:
                        acc = acc + ALPHA * buf1[i, sl]
                    buf1[i, sl] = acc
                return 0
            lax.fori_loop(0, UCH, _ubody, 0)
            if is_last:
                wd = pltpu.async_copy(buf1.at[pl.ds(0, UCH)],
                                      z_out.at[pl.ds(flat, UCH)], ss0)
            else:
                wd = pltpu.async_copy(buf1.at[pl.ds(0, UCH)],
                                      hn.at[pl.ds(flat, UCH)], ss0)
                _zero_agg_chunk(r0, sync=False, sem=ss1, out=zds)
        wd.wait()
        for zd in zds:
            zd.wait()

    def _layer(k, _):
        _gather_scatter()
        plsc.subcore_barrier()

        @pl.when(k < K_LAYERS - 1)
        def _mid():
            _update(False)

        @pl.when(k == K_LAYERS - 1)
        def _last():
            _update(True)
        plsc.subcore_barrier()
        return 0
    lax.fori_loop(0, K_LAYERS, _layer, 0)


def _sc_propagate(h_pad, srcidx, dstidx):
    f32 = jnp.float32
    mesh = plsc.VectorSubcoreMesh(core_axis_name="c", subcore_axis_name="s")
    kfn = pl.kernel(
        _sc_propagate_body,
        out_type=[
            jax.ShapeDtypeStruct((2 * NPAD, D), f32),
            jax.ShapeDtypeStruct((2 * NPAD, D), f32),
            jax.ShapeDtypeStruct((2 * NPAD, D), f32),
            jax.ShapeDtypeStruct((2 * NPAD, 16), f32),
            jax.ShapeDtypeStruct((2 * NPAD, 16), f32),
            jax.ShapeDtypeStruct((2 * NPAD, 16), f32),
        ],
        mesh=mesh,
        scratch_types=[
            pltpu.VMEM_SHARED((NPAD, D), f32),
            pltpu.VMEM((IB, CH), jnp.int32),
            pltpu.VMEM((IB, CH), jnp.int32),
            pltpu.VMEM((IB, CH), jnp.int32),
            pltpu.VMEM((IB, CH), jnp.int32),
            pltpu.VMEM((CH, D), f32),
            pltpu.VMEM((CH, D), f32),
            pltpu.VMEM((CH, D), f32),
            pltpu.VMEM((UCH, 16), f32),
            pltpu.VMEM((UCH, 16), f32),
            pltpu.SemaphoreType.DMA,
            pltpu.SemaphoreType.DMA,
            pltpu.SemaphoreType.DMA,
            pltpu.SemaphoreType.DMA,
            pltpu.SemaphoreType.DMA,
            pltpu.SemaphoreType.DMA,
            pltpu.SemaphoreType.DMA,
            pltpu.SemaphoreType.DMA,
        ],
    )
    z, _, _, _, _, _ = kfn(h_pad, srcidx, dstidx)
    return z


def _prep_edges(edge_index, graph_id):
    src = edge_index[0].astype(jnp.int32) + graph_id * NPAD
    dst = edge_index[1].astype(jnp.int32)
    pad = ECH_PF * CH - EPT
    src = jnp.pad(src.reshape(NTILES, EPT), ((0, 0), (0, pad)),
                  constant_values=graph_id * NPAD + N)
    dst = jnp.pad(dst.reshape(NTILES, EPT), ((0, 0), (0, pad)),
                  constant_values=N)
    return src.reshape(NTILES, ECH_PF, CH), dst.reshape(NTILES, ECH_PF, CH)


def _tc_matmul(x_pad, Wt):
    def body(x_ref, w_ref, o_ref):
        o_ref[...] = jnp.dot(x_ref[...], w_ref[...],
                             preferred_element_type=jnp.float32)
    return pl.pallas_call(
        body, out_shape=jax.ShapeDtypeStruct((NPAD, D), jnp.float32),
    )(x_pad, Wt)


def _tc_attention_beta(z_flat, Wa1, ba1_2d, Wa2):
    def body(z_ref, wa1_ref, ba1_ref, wa2_ref, beta_ref):
        wa1 = wa1_ref[...]
        ba1 = ba1_ref[...]
        wa2 = wa2_ref[...]
        mask = lax.broadcasted_iota(jnp.int32, (NPAD, 1), 0) < N
        s = []
        for m in range(2):
            zm = z_ref[m * NPAD:(m + 1) * NPAD, :]
            t = jnp.tanh(jnp.dot(zm, wa1, preferred_element_type=jnp.float32)
                         + ba1)
            t = jnp.dot(t, wa2, preferred_element_type=jnp.float32)
            s.append(jnp.sum(jnp.where(mask, t, 0.0)) / N)
        mx = jnp.maximum(s[0], s[1])
        e0 = jnp.exp(s[0] - mx)
        e1 = jnp.exp(s[1] - mx)
        den = e0 + e1
        ones = jnp.ones((1, D), jnp.float32)
        beta_ref[0:1, :] = (e0 / den) * ones
        beta_ref[1:2, :] = (e1 / den) * ones
    return pl.pallas_call(
        body, out_shape=jax.ShapeDtypeStruct((2, D), jnp.float32),
    )(z_flat, Wa1, ba1_2d, Wa2)


def _tc_combine(z_flat, beta, Wp, bp_2d):
    def body(z_ref, beta_ref, wp_ref, bp_ref, h_ref, lg_ref):
        h = (z_ref[0:NPAD, :] * beta_ref[0:1, :]
             + z_ref[NPAD:2 * NPAD, :] * beta_ref[1:2, :])
        h_ref[...] = h
        lg_ref[...] = jnp.dot(h, wp_ref[...],
                              preferred_element_type=jnp.float32) + bp_ref[...]
    return pl.pallas_call(
        body,
        out_shape=[
            jax.ShapeDtypeStruct((NPAD, D), jnp.float32),
            jax.ShapeDtypeStruct((NPAD, D_OUT), jnp.float32),
        ],
    )(z_flat, beta, Wp, bp_2d)


def kernel(x, edge_index0, edge_index1, Wt, Wa1, ba1, Wa2, Wp, bp):
    x_pad = jnp.pad(x, ((0, NPAD - N), (0, 0)))
    h_pad = _tc_matmul(x_pad, Wt)

    s0, d0 = _prep_edges(edge_index0, 0)
    s1, d1 = _prep_edges(edge_index1, 1)
    srcidx = jnp.concatenate([s0, s1], axis=0)
    dstidx = jnp.concatenate([d0, d1], axis=0)

    z_flat = _sc_propagate(h_pad, srcidx, dstidx)

    beta = _tc_attention_beta(z_flat, Wa1, ba1.reshape(1, D), Wa2)
    h_out, logits = _tc_combine(z_flat, beta, Wp, bp.reshape(1, D_OUT))
    return (logits[:N], h_out[:N])
